# Initial kernel scaffold; baseline (speedup 1.0000x reference)
import jax
import jax.numpy as jnp
from jax.experimental import pallas as pl


def kernel(hidden_states, rms_weight, gate_w, w1, w2, w3):
    raise NotImplementedError("write your pallas kernel here")


# speedup vs baseline: 1.8279x; 1.8279x over previous
import functools

import jax
import jax.numpy as jnp
from jax import lax
from jax.experimental import pallas as pl
from jax.experimental.pallas import tpu as pltpu
from jax.experimental.pallas import tpu_sc as plsc

H = 1024
FF = 3584
E = 8
TOPK = 2
EPS = 1e-06
T = 2048
A = T * TOPK
BLK = 256
NBLK = A // BLK + E
P = NBLK * BLK
LANES = 128

NTILES = 32
TPT = T // NTILES
CH = 32


def _routing_kernel(hs_ref, rmsw_ref, gw_ref, hsn_ref, pos_ref, tw_ref,
                    cnt_ref):
    x = hs_ref[...]
    var = jnp.mean(x * x, axis=1, keepdims=True)
    xn = x * lax.rsqrt(var + EPS) * rmsw_ref[...]
    hsn_ref[...] = xn

    logits = jnp.dot(xn, gw_ref[...], preferred_element_type=jnp.float32)
    lane = lax.broadcasted_iota(jnp.int32, (T, LANES), 1)
    valid = lane < E
    logits = jnp.where(valid, logits, -1e30)
    m = jnp.max(logits, axis=1, keepdims=True)
    p = jnp.where(valid, jnp.exp(logits - m), 0.0)
    sc = p / jnp.sum(p, axis=1, keepdims=True)

    m1 = jnp.max(sc, axis=1, keepdims=True)
    i1 = jnp.min(jnp.where((sc == m1) & valid, lane, LANES), axis=1,
                 keepdims=True)
    oh1 = lane == i1
    sc2 = jnp.where(oh1, -1.0, sc)
    m2 = jnp.max(sc2, axis=1, keepdims=True)
    i2 = jnp.min(jnp.where((sc2 == m2) & valid, lane, LANES), axis=1,
                 keepdims=True)
    oh2 = lane == i2
    ssum = m1 + m2
    tw_ref[...] = jnp.where(lane == 0, m1 / ssum,
                            jnp.where(lane == 1, m2 / ssum, 0.0))

    oh = (oh1 | oh2).astype(jnp.int32)
    c = oh
    d = 1
    while d < T:
        c = c + jnp.concatenate(
            [jnp.zeros((d, LANES), jnp.int32), c[:T - d]], axis=0)
        d *= 2
    cex = c - oh
    counts = c[T - 1:T, :]

    nb = jnp.where(lane[0:1, :] < E,
                   jnp.right_shift(counts + (BLK - 1), 8), 0)
    row = lax.broadcasted_iota(jnp.int32, (LANES, LANES), 0)
    col = lax.broadcasted_iota(jnp.int32, (LANES, LANES), 1)
    tri = (row < col).astype(jnp.float32)
    nb8 = jnp.broadcast_to(nb.astype(jnp.float32), (8, LANES))
    blk_base = jnp.dot(nb8, tri, preferred_element_type=jnp.float32)
    pad_base = (blk_base[0:1, :] * BLK).astype(jnp.int32)

    rank1 = jnp.sum(jnp.where(oh1, cex, 0), axis=1, keepdims=True)
    rank2 = jnp.sum(jnp.where(oh2, cex, 0), axis=1, keepdims=True)
    base1 = jnp.sum(jnp.where(oh1, pad_base, 0), axis=1, keepdims=True)
    base2 = jnp.sum(jnp.where(oh2, pad_base, 0), axis=1, keepdims=True)
    pos_ref[...] = jnp.where(lane == 0, base1 + rank1,
                             jnp.where(lane == 1, base2 + rank2, 0))
    cnt_ref[...] = jnp.broadcast_to(counts, (8, LANES))


def _routing(hs, rms_weight, gate_w_pad):
    return pl.pallas_call(
        _routing_kernel,
        out_shape=(
            jax.ShapeDtypeStruct((T, H), jnp.float32),
            jax.ShapeDtypeStruct((T, LANES), jnp.int32),
            jax.ShapeDtypeStruct((T, LANES), jnp.float32),
            jax.ShapeDtypeStruct((8, LANES), jnp.int32),
        ),
    )(hs, rms_weight, gate_w_pad)


def _sc_scatter_body(hsn, pos, xpad, pos_v, rows_v, sem):
    wid = lax.axis_index("s") * 2 + lax.axis_index("c")
    for ch in range(TPT // CH):
        tbase = wid * TPT + ch * CH
        pltpu.sync_copy(hsn.at[pl.ds(tbase, CH)], rows_v)
        for k in range(TOPK):
            pltpu.sync_copy(pos.at[k, pl.ds(tbase, CH)], pos_v)
            pltpu.async_copy(rows_v, xpad.at[pos_v], sem).wait()


def _scatter_sc(hsn, pos_slots):
    mesh = plsc.VectorSubcoreMesh(core_axis_name="c", subcore_axis_name="s")
    f = functools.partial(
        pl.kernel,
        mesh=mesh,
        out_type=jax.ShapeDtypeStruct((P, H), jnp.float32),
        scratch_types=[
            pltpu.VMEM((CH,), jnp.int32),
            pltpu.VMEM((CH, H), jnp.float32),
            pltpu.SemaphoreType.DMA,
        ],
    )(_sc_scatter_body)
    return f(hsn, pos_slots)


def _ffn_kernel(be_ref, act_ref, x_ref, w1_ref, w3_ref, w2_ref, o_ref):
    b = pl.program_id(0)

    @pl.when(act_ref[b] == 1)
    def _():
        x = x_ref[...].astype(jnp.bfloat16)
        h1 = jnp.dot(x, w1_ref[0], preferred_element_type=jnp.float32)
        h3 = jnp.dot(x, w3_ref[0], preferred_element_type=jnp.float32)
        a = (h1 * lax.logistic(h1)) * h3
        o_ref[...] = jnp.dot(a.astype(jnp.bfloat16), w2_ref[0],
                             preferred_element_type=jnp.float32)


def _ffn_grouped(blk_e, blk_act, x_pad, w1b, w3b, w2b):
    grid_spec = pltpu.PrefetchScalarGridSpec(
        num_scalar_prefetch=2,
        grid=(NBLK,),
        in_specs=[
            pl.BlockSpec((BLK, H), lambda b, be, act: (b, 0)),
            pl.BlockSpec((1, H, FF), lambda b, be, act: (be[b], 0, 0)),
            pl.BlockSpec((1, H, FF), lambda b, be, act: (be[b], 0, 0)),
            pl.BlockSpec((1, FF, H), lambda b, be, act: (be[b], 0, 0)),
        ],
        out_specs=pl.BlockSpec((BLK, H), lambda b, be, act: (b, 0)),
    )
    return pl.pallas_call(
        _ffn_kernel,
        grid_spec=grid_spec,
        out_shape=jax.ShapeDtypeStruct((P, H), jnp.float32),
    )(blk_e, blk_act, x_pad, w1b, w3b, w2b)


def _sc_gather_body(outpad, pos, gat, pos_v, rows_v, sem):
    wid = lax.axis_index("s") * 2 + lax.axis_index("c")
    for k in range(TOPK):
        for ch in range(TPT // CH):
            tbase = wid * TPT + ch * CH
            pltpu.sync_copy(pos.at[k, pl.ds(tbase, CH)], pos_v)
            pltpu.async_copy(outpad.at[pos_v], rows_v, sem).wait()
            pltpu.sync_copy(rows_v, gat.at[k, pl.ds(tbase, CH)])


def _gather_sc(out_pad, pos_slots):
    mesh = plsc.VectorSubcoreMesh(core_axis_name="c", subcore_axis_name="s")
    f = functools.partial(
        pl.kernel,
        mesh=mesh,
        out_type=jax.ShapeDtypeStruct((TOPK, T, H), jnp.float32),
        scratch_types=[
            pltpu.VMEM((CH,), jnp.int32),
            pltpu.VMEM((CH, H), jnp.float32),
            pltpu.SemaphoreType.DMA,
        ],
    )(_sc_gather_body)
    return f(out_pad, pos_slots)


def _combine_kernel(inp_ref, g_ref, tw_ref, o_ref):
    tw0 = tw_ref[:, 0:1]
    tw1 = tw_ref[:, 1:2]
    o_ref[...] = inp_ref[...] + tw0 * g_ref[0] + tw1 * g_ref[1]


def _combine(inp, gathered, tw):
    nb = 8
    tb = T // nb
    return pl.pallas_call(
        _combine_kernel,
        grid=(nb,),
        in_specs=[
            pl.BlockSpec((tb, H), lambda i: (i, 0)),
            pl.BlockSpec((TOPK, tb, H), lambda i: (0, i, 0)),
            pl.BlockSpec((tb, LANES), lambda i: (i, 0)),
        ],
        out_specs=pl.BlockSpec((tb, H), lambda i: (i, 0)),
        out_shape=jax.ShapeDtypeStruct((T, H), jnp.float32),
    )(inp, gathered, tw)


def kernel(hidden_states, rms_weight, gate_w, w1, w2, w3):
    b, s, h = hidden_states.shape
    hs = hidden_states.reshape(T, H)
    rmsw = rms_weight.reshape(1, H)
    gwp = jnp.pad(gate_w, ((0, 0), (0, LANES - E)))

    hsn, pos128, tw128, cnt = _routing(hs, rmsw, gwp)
    pos_slots = jnp.transpose(pos128[:, :TOPK]).astype(jnp.int32)

    counts = cnt[0, :E]
    nblocks = (counts + (BLK - 1)) // BLK
    blk_base = jnp.cumsum(nblocks) - nblocks
    total = jnp.sum(nblocks)
    bi = jnp.arange(NBLK, dtype=jnp.int32)
    owner = jnp.sum((blk_base[None, :] <= bi[:, None]).astype(jnp.int32),
                    axis=1) - 1
    act = (bi < total).astype(jnp.int32)
    e_last = jnp.sum(jnp.where(bi == total - 1, owner, 0))
    blk_e = jnp.where(act == 1, owner, e_last).astype(jnp.int32)

    x_pad = _scatter_sc(hsn, pos_slots)
    out_pad = _ffn_grouped(blk_e, act, x_pad,
                           w1.astype(jnp.bfloat16),
                           w3.astype(jnp.bfloat16),
                           w2.astype(jnp.bfloat16))
    gathered = _gather_sc(out_pad, pos_slots)
    final = _combine(hs, gathered, tw128)
    return final.reshape(b, s, h)


# speedup vs baseline: 2.1933x; 1.1999x over previous
import functools

import jax
import jax.numpy as jnp
from jax import lax
from jax.experimental import pallas as pl
from jax.experimental.pallas import tpu as pltpu
from jax.experimental.pallas import tpu_sc as plsc

H = 1024
FF = 3584
E = 8
TOPK = 2
EPS = 1e-06
T = 2048
A = T * TOPK
BLK = 256
NBLK = A // BLK + E
P = NBLK * BLK
LANES = 128

NTILES = 32
TPT = T // NTILES
CH = 32


def _routing_kernel(hs_ref, rmsw_ref, gw_ref, hsn_ref, pos_ref, tw_ref,
                    cnt_ref):
    x = hs_ref[...]
    var = jnp.mean(x * x, axis=1, keepdims=True)
    xn = x * lax.rsqrt(var + EPS) * rmsw_ref[...]
    hsn_ref[...] = xn

    logits = jnp.dot(xn, gw_ref[...], preferred_element_type=jnp.float32)
    lane = lax.broadcasted_iota(jnp.int32, (T, LANES), 1)
    valid = lane < E
    logits = jnp.where(valid, logits, -1e30)
    m = jnp.max(logits, axis=1, keepdims=True)
    p = jnp.where(valid, jnp.exp(logits - m), 0.0)
    sc = p / jnp.sum(p, axis=1, keepdims=True)

    m1 = jnp.max(sc, axis=1, keepdims=True)
    i1 = jnp.min(jnp.where((sc == m1) & valid, lane, LANES), axis=1,
                 keepdims=True)
    oh1 = lane == i1
    sc2 = jnp.where(oh1, -1.0, sc)
    m2 = jnp.max(sc2, axis=1, keepdims=True)
    i2 = jnp.min(jnp.where((sc2 == m2) & valid, lane, LANES), axis=1,
                 keepdims=True)
    oh2 = lane == i2
    ssum = m1 + m2
    tw_ref[...] = jnp.where(lane == 0, m1 / ssum,
                            jnp.where(lane == 1, m2 / ssum, 0.0))

    oh = (oh1 | oh2).astype(jnp.int32)
    c = oh
    d = 1
    while d < T:
        c = c + jnp.concatenate(
            [jnp.zeros((d, LANES), jnp.int32), c[:T - d]], axis=0)
        d *= 2
    cex = c - oh
    counts = c[T - 1:T, :]

    nb = jnp.where(lane[0:1, :] < E,
                   jnp.right_shift(counts + (BLK - 1), 8), 0)
    row = lax.broadcasted_iota(jnp.int32, (LANES, LANES), 0)
    col = lax.broadcasted_iota(jnp.int32, (LANES, LANES), 1)
    tri = (row < col).astype(jnp.float32)
    nb8 = jnp.broadcast_to(nb.astype(jnp.float32), (8, LANES))
    blk_base = jnp.dot(nb8, tri, preferred_element_type=jnp.float32)
    pad_base = (blk_base[0:1, :] * BLK).astype(jnp.int32)

    rank1 = jnp.sum(jnp.where(oh1, cex, 0), axis=1, keepdims=True)
    rank2 = jnp.sum(jnp.where(oh2, cex, 0), axis=1, keepdims=True)
    base1 = jnp.sum(jnp.where(oh1, pad_base, 0), axis=1, keepdims=True)
    base2 = jnp.sum(jnp.where(oh2, pad_base, 0), axis=1, keepdims=True)
    pos_ref[...] = jnp.where(lane == 0, base1 + rank1,
                             jnp.where(lane == 1, base2 + rank2, 0))
    cnt_ref[...] = jnp.broadcast_to(counts, (8, LANES))


def _routing(hs, rms_weight, gate_w_pad):
    return pl.pallas_call(
        _routing_kernel,
        out_shape=(
            jax.ShapeDtypeStruct((T, H), jnp.float32),
            jax.ShapeDtypeStruct((T, LANES), jnp.int32),
            jax.ShapeDtypeStruct((T, LANES), jnp.float32),
            jax.ShapeDtypeStruct((8, LANES), jnp.int32),
        ),
    )(hs, rms_weight, gate_w_pad)


def _sc_scatter_body(hsn, pos, xpad, pos_v, rows_v, sem):
    wid = lax.axis_index("s") * 2 + lax.axis_index("c")
    for ch in range(TPT // CH):
        tbase = wid * TPT + ch * CH
        pltpu.sync_copy(hsn.at[pl.ds(tbase, CH)], rows_v)
        for k in range(TOPK):
            pltpu.sync_copy(pos.at[k, pl.ds(tbase, CH)], pos_v)
            pltpu.async_copy(rows_v, xpad.at[pos_v], sem).wait()


def _scatter_sc(hsn, pos_slots):
    mesh = plsc.VectorSubcoreMesh(core_axis_name="c", subcore_axis_name="s")
    f = functools.partial(
        pl.kernel,
        mesh=mesh,
        out_type=jax.ShapeDtypeStruct((P, H), jnp.float32),
        scratch_types=[
            pltpu.VMEM((CH,), jnp.int32),
            pltpu.VMEM((CH, H), jnp.float32),
            pltpu.SemaphoreType.DMA,
        ],
    )(_sc_scatter_body)
    return f(hsn, pos_slots)


FT = 2
FFT = FF // FT


def _gateup_kernel(be_ref, act_ref, x_ref, w1_ref, w3_ref, a_ref):
    b = pl.program_id(1)

    @pl.when(act_ref[b] == 1)
    def _():
        x = x_ref[...].astype(jnp.bfloat16)
        h1 = jnp.dot(x, w1_ref[0].astype(jnp.bfloat16),
                     preferred_element_type=jnp.float32)
        h3 = jnp.dot(x, w3_ref[0].astype(jnp.bfloat16),
                     preferred_element_type=jnp.float32)
        a_ref[...] = ((h1 * lax.logistic(h1)) * h3).astype(jnp.bfloat16)


def _down_kernel(be_ref, act_ref, a_ref, w2_ref, o_ref):
    b = pl.program_id(0)

    @pl.when(act_ref[b] == 1)
    def _():
        o_ref[...] = jnp.dot(a_ref[...], w2_ref[0].astype(jnp.bfloat16),
                             preferred_element_type=jnp.float32)


def _ffn_grouped(blk_e, blk_act, x_pad, w1, w3, w2):
    a = pl.pallas_call(
        _gateup_kernel,
        grid_spec=pltpu.PrefetchScalarGridSpec(
            num_scalar_prefetch=2,
            grid=(FT, NBLK),
            in_specs=[
                pl.BlockSpec((BLK, H), lambda f, b, be, act: (b, 0)),
                pl.BlockSpec((1, H, FFT), lambda f, b, be, act: (be[b], 0, f)),
                pl.BlockSpec((1, H, FFT), lambda f, b, be, act: (be[b], 0, f)),
            ],
            out_specs=pl.BlockSpec((BLK, FFT), lambda f, b, be, act: (b, f)),
        ),
        out_shape=jax.ShapeDtypeStruct((P, FF), jnp.bfloat16),
    )(blk_e, blk_act, x_pad, w1, w3)
    return pl.pallas_call(
        _down_kernel,
        grid_spec=pltpu.PrefetchScalarGridSpec(
            num_scalar_prefetch=2,
            grid=(NBLK,),
            in_specs=[
                pl.BlockSpec((BLK, FF), lambda b, be, act: (b, 0)),
                pl.BlockSpec((1, FF, H), lambda b, be, act: (be[b], 0, 0)),
            ],
            out_specs=pl.BlockSpec((BLK, H), lambda b, be, act: (b, 0)),
        ),
        out_shape=jax.ShapeDtypeStruct((P, H), jnp.float32),
    )(blk_e, blk_act, a, w2)


def _sc_gather_body(outpad, pos, gat, pos_v, rows_v, sem):
    wid = lax.axis_index("s") * 2 + lax.axis_index("c")
    for k in range(TOPK):
        for ch in range(TPT // CH):
            tbase = wid * TPT + ch * CH
            pltpu.sync_copy(pos.at[k, pl.ds(tbase, CH)], pos_v)
            pltpu.async_copy(outpad.at[pos_v], rows_v, sem).wait()
            pltpu.sync_copy(rows_v, gat.at[k, pl.ds(tbase, CH)])


def _gather_sc(out_pad, pos_slots):
    mesh = plsc.VectorSubcoreMesh(core_axis_name="c", subcore_axis_name="s")
    f = functools.partial(
        pl.kernel,
        mesh=mesh,
        out_type=jax.ShapeDtypeStruct((TOPK, T, H), jnp.float32),
        scratch_types=[
            pltpu.VMEM((CH,), jnp.int32),
            pltpu.VMEM((CH, H), jnp.float32),
            pltpu.SemaphoreType.DMA,
        ],
    )(_sc_gather_body)
    return f(out_pad, pos_slots)


def _combine_kernel(inp_ref, g_ref, tw_ref, o_ref):
    tw0 = tw_ref[:, 0:1]
    tw1 = tw_ref[:, 1:2]
    o_ref[...] = inp_ref[...] + tw0 * g_ref[0] + tw1 * g_ref[1]


def _combine(inp, gathered, tw):
    nb = 8
    tb = T // nb
    return pl.pallas_call(
        _combine_kernel,
        grid=(nb,),
        in_specs=[
            pl.BlockSpec((tb, H), lambda i: (i, 0)),
            pl.BlockSpec((TOPK, tb, H), lambda i: (0, i, 0)),
            pl.BlockSpec((tb, LANES), lambda i: (i, 0)),
        ],
        out_specs=pl.BlockSpec((tb, H), lambda i: (i, 0)),
        out_shape=jax.ShapeDtypeStruct((T, H), jnp.float32),
    )(inp, gathered, tw)


def kernel(hidden_states, rms_weight, gate_w, w1, w2, w3):
    b, s, h = hidden_states.shape
    hs = hidden_states.reshape(T, H)
    rmsw = rms_weight.reshape(1, H)
    gwp = jnp.pad(gate_w, ((0, 0), (0, LANES - E)))

    hsn, pos128, tw128, cnt = _routing(hs, rmsw, gwp)
    pos_slots = jnp.transpose(pos128[:, :TOPK]).astype(jnp.int32)

    counts = cnt[0, :E]
    nblocks = (counts + (BLK - 1)) // BLK
    blk_base = jnp.cumsum(nblocks) - nblocks
    total = jnp.sum(nblocks)
    bi = jnp.arange(NBLK, dtype=jnp.int32)
    owner = jnp.sum((blk_base[None, :] <= bi[:, None]).astype(jnp.int32),
                    axis=1) - 1
    act = (bi < total).astype(jnp.int32)
    e_last = jnp.sum(jnp.where(bi == total - 1, owner, 0))
    blk_e = jnp.where(act == 1, owner, e_last).astype(jnp.int32)

    x_pad = _scatter_sc(hsn, pos_slots)
    out_pad = _ffn_grouped(blk_e, act, x_pad, w1, w3, w2)
    gathered = _gather_sc(out_pad, pos_slots)
    final = _combine(hs, gathered, tw128)
    return final.reshape(b, s, h)
